# baseline (speedup 1.0000x reference)
import jax
import jax.numpy as jnp
from jax import lax
from jax.experimental import pallas as pl
from jax.experimental.pallas import tpu as pltpu
from jax.experimental.pallas import tpu_sc as plsc

DIM = 64
B = 4096
S = 200
NC = 2
NS = 16
NW = NC * NS
BB = B // NW


def _emb_body(idsT_hbm, table2_hbm, pos_hbm, out_hbm,
              ids_v, q_v, h_v, rows_v, stage_v, pos_v, sem):
    cid = lax.axis_index("c")
    sid = lax.axis_index("s")
    wid = sid * NC + cid
    b0 = wid * BB

    @pl.loop(0, S // 8)
    def _sblock(sb):
        pltpu.sync_copy(idsT_hbm.at[pl.ds(sb * 8, 8), pl.ds(b0, BB)], ids_v)
        pltpu.sync_copy(pos_hbm.at[pl.ds(sb * 8, 8)], pos_v)

        for j in range(8):
            for g in range(BB // 16):
                v = ids_v[j, pl.ds(g * 16, 16)]
                q_v[pl.ds(g * 16, 16)] = jnp.right_shift(v, 1)
                h_v[pl.ds(g * 16, 16)] = jnp.bitwise_and(v, 1)

            pltpu.async_copy(table2_hbm.at[q_v], rows_v, sem).wait()

            @pl.loop(0, DIM)
            def _drow(d):
                dvec = jnp.full((16,), 0, jnp.int32) + d
                jvec = jnp.full((16,), j, jnp.int32)
                pvec = plsc.load_gather(pos_v, [jvec, dvec])
                for g in range(BB // 16):
                    rid = jax.lax.broadcasted_iota(jnp.int32, (16,), 0) + g * 16
                    colv = h_v[pl.ds(g * 16, 16)] * DIM + d
                    vals = plsc.load_gather(rows_v, [rid, colv])
                    stage_v[d, pl.ds(g * 16, 16)] = vals + pvec

            pltpu.sync_copy(
                stage_v,
                out_hbm.at[sb * 8 + j, :, pl.ds(b0, BB)],
            )


def kernel(token_ids, token_table, pos_table):
    idsT = token_ids.astype(jnp.int32).T
    table2 = token_table.reshape(500000, 128)
    k = pl.kernel(
        _emb_body,
        out_type=jax.ShapeDtypeStruct((S, DIM, B), jnp.float32),
        mesh=plsc.VectorSubcoreMesh(core_axis_name="c", subcore_axis_name="s"),
        compiler_params=pltpu.CompilerParams(
            use_tc_tiling_on_sc=True, needs_layout_passes=False),
        scratch_types=[
            pltpu.VMEM((8, BB), jnp.int32),
            pltpu.VMEM((BB,), jnp.int32),
            pltpu.VMEM((BB,), jnp.int32),
            pltpu.VMEM((BB, 2 * DIM), jnp.float32),
            pltpu.VMEM((DIM, BB), jnp.float32),
            pltpu.VMEM((8, DIM), jnp.float32),
            pltpu.SemaphoreType.DMA,
        ],
    )
    out = k(idsT, table2, pos_table[:S])
    return out.transpose(2, 0, 1)


# speedup vs baseline: 1.1370x; 1.1370x over previous
import jax
import jax.numpy as jnp
from jax import lax
from jax.experimental import pallas as pl
from jax.experimental.pallas import tpu as pltpu
from jax.experimental.pallas import tpu_sc as plsc

DIM = 64
B = 4096
S = 200
NC = 2
NS = 16
NW = NC * NS
BB = B // NW


def _emb_body(idsT_hbm, table2_hbm, pos_hbm, out_hbm,
              ids_v, pos_v, q0, q1, h0, h1, rows0, rows1, st0, st1,
              semg0, semg1, semo0, semo1):
    cid = lax.axis_index("c")
    sid = lax.axis_index("s")
    wid = sid * NC + cid
    b0 = wid * BB

    pltpu.sync_copy(idsT_hbm.at[:, pl.ds(b0, BB)], ids_v)
    pltpu.sync_copy(pos_hbm, pos_v)

    def qh(s, q_v, h_v):
        for g in range(BB // 16):
            v = ids_v[s, pl.ds(g * 16, 16)]
            q_v[pl.ds(g * 16, 16)] = jnp.right_shift(v, 1)
            h_v[pl.ds(g * 16, 16)] = jnp.bitwise_and(v, 1)

    def gather_start(q_v, rows, sem):
        return pltpu.async_copy(table2_hbm.at[q_v], rows, sem)

    def gather_wait(q_v, rows, sem):
        pltpu.make_async_copy(table2_hbm.at[q_v], rows, sem).wait()

    def transpose_add(s, h_v, rows, stage):
        @pl.loop(0, DIM)
        def _drow(d):
            dvec = jnp.full((16,), 0, jnp.int32) + d
            svec = jnp.full((16,), 0, jnp.int32) + s
            pvec = plsc.load_gather(pos_v, [svec, dvec])
            for g in range(BB // 16):
                rid = lax.broadcasted_iota(jnp.int32, (16,), 0) + g * 16
                colv = h_v[pl.ds(g * 16, 16)] * DIM + d
                vals = plsc.load_gather(rows, [rid, colv])
                stage[d, pl.ds(g * 16, 16)] = vals + pvec

    def out_start(s, stage, sem):
        return pltpu.async_copy(stage, out_hbm.at[s, :, pl.ds(b0, BB)], sem)

    def out_wait(s, stage, sem):
        pltpu.make_async_copy(stage, out_hbm.at[s, :, pl.ds(b0, BB)], sem).wait()

    qh(0, q0, h0)
    gather_start(q0, rows0, semg0)

    @pl.loop(0, S, step=2)
    def _sloop(i):
        qh(i + 1, q1, h1)
        gather_wait(q0, rows0, semg0)
        gather_start(q1, rows1, semg1)

        @pl.when(i >= 2)
        def _():
            out_wait(i - 2, st0, semo0)

        transpose_add(i, h0, rows0, st0)
        out_start(i, st0, semo0)

        @pl.when(i + 2 < S)
        def _():
            qh(i + 2, q0, h0)

        gather_wait(q1, rows1, semg1)

        @pl.when(i + 2 < S)
        def _():
            gather_start(q0, rows0, semg0)

        @pl.when(i >= 2)
        def _():
            out_wait(i - 1, st1, semo1)

        transpose_add(i + 1, h1, rows1, st1)
        out_start(i + 1, st1, semo1)

    out_wait(S - 2, st0, semo0)
    out_wait(S - 1, st1, semo1)


def kernel(token_ids, token_table, pos_table):
    idsT = token_ids.astype(jnp.int32).T
    table2 = token_table.reshape(500000, 128)
    k = pl.kernel(
        _emb_body,
        out_type=jax.ShapeDtypeStruct((S, DIM, B), jnp.float32),
        mesh=plsc.VectorSubcoreMesh(core_axis_name="c", subcore_axis_name="s"),
        compiler_params=pltpu.CompilerParams(
            use_tc_tiling_on_sc=True, needs_layout_passes=False),
        scratch_types=[
            pltpu.VMEM((S, BB), jnp.int32),
            pltpu.VMEM((S, DIM), jnp.float32),
            pltpu.VMEM((BB,), jnp.int32),
            pltpu.VMEM((BB,), jnp.int32),
            pltpu.VMEM((BB,), jnp.int32),
            pltpu.VMEM((BB,), jnp.int32),
            pltpu.VMEM((BB, 2 * DIM), jnp.float32),
            pltpu.VMEM((BB, 2 * DIM), jnp.float32),
            pltpu.VMEM((DIM, BB), jnp.float32),
            pltpu.VMEM((DIM, BB), jnp.float32),
            pltpu.SemaphoreType.DMA,
            pltpu.SemaphoreType.DMA,
            pltpu.SemaphoreType.DMA,
            pltpu.SemaphoreType.DMA,
        ],
    )
    out = k(idsT, table2, pos_table[:S])
    return out.transpose(2, 0, 1)


# speedup vs baseline: 1.6660x; 1.4653x over previous
import jax
import jax.numpy as jnp
from jax import lax
from jax.experimental import pallas as pl
from jax.experimental.pallas import tpu as pltpu
from jax.experimental.pallas import tpu_sc as plsc

DIM = 64
B = 4096
S = 200
NC = 2
NS = 16
NW = NC * NS
RPW = B // NW
TPW = RPW * S


def _emb_body(ids_hbm, table2_hbm, pos_hbm, out_hbm,
              pos_v, idsA, idsB, q0, q1, rows0, rows1, st0, st1,
              semg0, semg1, semi0, semi1):
    cid = lax.axis_index("c")
    sid = lax.axis_index("s")
    wid = sid * NC + cid
    row0 = wid * RPW
    tok0 = wid * TPW

    pltpu.sync_copy(pos_hbm.at[pl.ds(0, S)], pos_v)

    def ids_start(s, ids_c, sem):
        return pltpu.async_copy(
            ids_hbm.at[pl.ds(tok0 + s * S, S)], ids_c.at[pl.ds(0, S)], sem)

    def ids_wait(ids_c, sem):
        pltpu.make_async_copy(
            ids_hbm.at[pl.ds(tok0, S)], ids_c.at[pl.ds(0, S)], sem).wait()

    def qcalc(ids_c, q_v):
        for g in range(13):
            v = ids_c[pl.ds(g * 16, 16)]
            q_v[pl.ds(g * 16, 16)] = jnp.right_shift(v, 1)

    def gather_start(q_v, rows, sem):
        return pltpu.async_copy(
            table2_hbm.at[q_v.at[pl.ds(0, S)]], rows, sem)

    def gather_wait(q_v, rows, sem):
        pltpu.make_async_copy(
            table2_hbm.at[q_v.at[pl.ds(0, S)]], rows, sem).wait()

    def select_add(ids_c, rows, stage):
        @pl.loop(0, S, unroll=4)
        def _tok(r):
            vvec = plsc.load_gather(ids_c, [jnp.full((16,), 0, jnp.int32) + r])
            hoff = jnp.bitwise_and(vvec, 1) * DIM
            rvec = jnp.full((16,), 0, jnp.int32) + r
            lane = lax.broadcasted_iota(jnp.int32, (16,), 0)
            for c in range(DIM // 16):
                vals = plsc.load_gather(rows, [rvec, hoff + lane + c * 16])
                stage[r, pl.ds(c * 16, 16)] = vals + pos_v[r, pl.ds(c * 16, 16)]

    def flush(s, stage):
        pltpu.sync_copy(stage, out_hbm.at[row0 + s])

    ids_start(0, idsA, semi0).wait()
    qcalc(idsA, q0)
    gather_start(q0, rows0, semg0)
    ids_start(1, idsB, semi1)

    @pl.loop(0, RPW, step=2)
    def _chunk_loop(i):
        @pl.when(i + 1 < RPW)
        def _():
            ids_wait(idsB, semi1)
            qcalc(idsB, q1)

        gather_wait(q0, rows0, semg0)

        @pl.when(i + 1 < RPW)
        def _():
            gather_start(q1, rows1, semg1)

        select_add(idsA, rows0, st0)
        flush(i, st0)

        @pl.when(i + 2 < RPW)
        def _():
            ids_start(i + 2, idsA, semi0)

        @pl.when(i + 2 < RPW)
        def _():
            ids_wait(idsA, semi0)
            qcalc(idsA, q0)

        gather_wait(q1, rows1, semg1)

        @pl.when(i + 2 < RPW)
        def _():
            gather_start(q0, rows0, semg0)

        select_add(idsB, rows1, st1)
        flush(i + 1, st1)

        @pl.when(i + 3 < RPW)
        def _():
            ids_start(i + 3, idsB, semi1)


def kernel(token_ids, token_table, pos_table):
    ids_flat = token_ids.astype(jnp.int32).reshape(B * S)
    table2 = token_table.reshape(500000, 128)
    k = pl.kernel(
        _emb_body,
        out_type=jax.ShapeDtypeStruct((B, S, DIM), jnp.float32),
        mesh=plsc.VectorSubcoreMesh(core_axis_name="c", subcore_axis_name="s"),
        compiler_params=pltpu.CompilerParams(
            use_tc_tiling_on_sc=True, needs_layout_passes=False),
        scratch_types=[
            pltpu.VMEM((S, DIM), jnp.float32),
            pltpu.VMEM((208,), jnp.int32),
            pltpu.VMEM((208,), jnp.int32),
            pltpu.VMEM((208,), jnp.int32),
            pltpu.VMEM((208,), jnp.int32),
            pltpu.VMEM((S, 2 * DIM), jnp.float32),
            pltpu.VMEM((S, 2 * DIM), jnp.float32),
            pltpu.VMEM((S, DIM), jnp.float32),
            pltpu.VMEM((S, DIM), jnp.float32),
            pltpu.SemaphoreType.DMA,
            pltpu.SemaphoreType.DMA,
            pltpu.SemaphoreType.DMA,
            pltpu.SemaphoreType.DMA,
        ],
    )
    return k(ids_flat, table2, pos_table)


# speedup vs baseline: 2.3575x; 1.4151x over previous
import jax
import jax.numpy as jnp
from jax import lax
from jax.experimental import pallas as pl
from jax.experimental.pallas import tpu as pltpu
from jax.experimental.pallas import tpu_sc as plsc

DIM = 64
B = 4096
S = 200
NC = 2
NS = 16
NW = NC * NS
RPW = B // NW
CHUNK = 2
CPW = RPW // CHUNK


def _emb_body(ids_hbm, table_hbm, pos_hbm, out_hbm,
              idx_all, pos_v, rows0, rows1, sem0, sem1):
    cid = lax.axis_index("c")
    sid = lax.axis_index("s")
    wid = sid * NC + cid
    row0 = wid * RPW

    pltpu.sync_copy(ids_hbm.at[pl.ds(row0, RPW)], idx_all)
    pltpu.sync_copy(pos_hbm.at[pl.ds(0, S)], pos_v.at[0])
    pltpu.sync_copy(pos_hbm.at[pl.ds(0, S)], pos_v.at[1])

    def gather_into(i, rows, sem):
        for b in range(CHUNK):
            pltpu.async_copy(
                table_hbm.at[idx_all.at[CHUNK * i + b]],
                rows.at[b],
                sem,
            )

    def wait_gather(rows, sem):
        for b in range(CHUNK):
            pltpu.make_async_copy(
                table_hbm.at[idx_all.at[b]],
                rows.at[b],
                sem,
            ).wait()

    def add_pos_and_flush(i, rows):
        for b in range(CHUNK):
            @pl.loop(0, S, unroll=8)
            def _add_loop(r):
                for c4 in range(DIM // 16):
                    plsc.addupdate(
                        rows.at[b, r, pl.ds(c4 * 16, 16)],
                        pos_v[b, r, pl.ds(c4 * 16, 16)],
                    )
        pltpu.sync_copy(rows, out_hbm.at[pl.ds(row0 + CHUNK * i, CHUNK)])

    gather_into(0, rows0, sem0)

    @pl.loop(0, CPW, step=2)
    def _chunk_loop(i):
        wait_gather(rows0, sem0)
        gather_into(i + 1, rows1, sem1)
        add_pos_and_flush(i, rows0)

        wait_gather(rows1, sem1)

        @pl.when(i + 2 < CPW)
        def _():
            gather_into(i + 2, rows0, sem0)

        add_pos_and_flush(i + 1, rows1)


def kernel(token_ids, token_table, pos_table):
    k = pl.kernel(
        _emb_body,
        out_type=jax.ShapeDtypeStruct((B, S, DIM), jnp.float32),
        mesh=plsc.VectorSubcoreMesh(core_axis_name="c", subcore_axis_name="s"),
        compiler_params=pltpu.CompilerParams(use_tc_tiling_on_sc=False),
        scratch_types=[
            pltpu.VMEM((RPW, S), jnp.int32),
            pltpu.VMEM((CHUNK, S, DIM), jnp.float32),
            pltpu.VMEM((CHUNK, S, DIM), jnp.float32),
            pltpu.VMEM((CHUNK, S, DIM), jnp.float32),
            pltpu.SemaphoreType.DMA,
            pltpu.SemaphoreType.DMA,
        ],
    )
    return k(token_ids.astype(jnp.int32), token_table, pos_table)
